# baseline (speedup 1.0000x reference)
import functools

import jax
import jax.numpy as jnp
from jax import lax
from jax.experimental import pallas as pl
from jax.experimental.pallas import tpu as pltpu
from jax.experimental.pallas import tpu_sc as plsc

DIM = 4096
ROWS = 2 * 4096
NC = 2
NS = 16
L = 16
NW = NC * NS
ROWS_PER_W = ROWS // NW
R = 4
CHUNKS = ROWS_PER_W // R
JV = DIM // L

_mesh = plsc.VectorSubcoreMesh(core_axis_name="c", subcore_axis_name="s")


@functools.partial(
    pl.kernel,
    mesh=_mesh,
    compiler_params=pltpu.CompilerParams(needs_layout_passes=False),
    out_type=jax.ShapeDtypeStruct((ROWS * DIM,), jnp.float32),
    scratch_types=[
        pltpu.VMEM((DIM,), jnp.int32),
        pltpu.VMEM((DIM,), jnp.float32),
        pltpu.VMEM((DIM,), jnp.float32),
        pltpu.VMEM((R * DIM,), jnp.float32),
        pltpu.VMEM((R * DIM,), jnp.float32),
    ],
)
def _dpd_sc(x_hbm, s1_hbm, s2_hbm, perm_hbm, out_hbm,
            perm_v, s1_v, s_v, in_v, out_v):
    wid = lax.axis_index("s") * NC + lax.axis_index("c")
    base = wid * (ROWS_PER_W * DIM)

    pltpu.sync_copy(perm_hbm, perm_v)
    pltpu.sync_copy(s1_hbm, s1_v)
    pltpu.sync_copy(s2_hbm, s_v)

    def sbody(j, carry):
        sl = pl.ds(j * L, L)
        pv = perm_v[sl]
        s_v[sl] = plsc.load_gather(s1_v, [pv]) * s_v[sl]
        return carry

    lax.fori_loop(0, JV, sbody, 0)

    def cbody(c, carry):
        off = base + c * (R * DIM)
        pltpu.sync_copy(x_hbm.at[pl.ds(off, R * DIM)], in_v)

        def jbody(j, inner):
            sl = pl.ds(j * L, L)
            pv = perm_v[sl]
            sv = s_v[sl]
            for r in range(R):
                g = plsc.load_gather(in_v, [pv + (r * DIM)])
                out_v[pl.ds(r * DIM + j * L, L)] = g * sv
            return inner

        lax.fori_loop(0, JV, jbody, 0)
        pltpu.sync_copy(out_v, out_hbm.at[pl.ds(off, R * DIM)])
        return carry

    lax.fori_loop(0, CHUNKS, cbody, 0)


def kernel(x, sign1, sign2, perm):
    out = _dpd_sc(x.reshape(-1), sign1, sign2, perm.astype(jnp.int32))
    return out.reshape(x.shape)


# speedup vs baseline: 1.1985x; 1.1985x over previous
import functools

import jax
import jax.numpy as jnp
from jax import lax
from jax.experimental import pallas as pl
from jax.experimental.pallas import tpu as pltpu
from jax.experimental.pallas import tpu_sc as plsc

DIM = 4096
ROWS = 2 * 4096
NC = 2
NS = 16
L = 16
NW = NC * NS
ROWS_PER_W = ROWS // NW
R = 4
CH = R * DIM
CHUNKS = ROWS_PER_W // R
JV = DIM // L

_mesh = plsc.VectorSubcoreMesh(core_axis_name="c", subcore_axis_name="s")


@functools.partial(
    pl.kernel,
    mesh=_mesh,
    compiler_params=pltpu.CompilerParams(needs_layout_passes=False),
    out_type=jax.ShapeDtypeStruct((ROWS * DIM,), jnp.float32),
    scratch_types=[
        pltpu.VMEM((DIM,), jnp.int32),
        pltpu.VMEM((DIM,), jnp.float32),
        pltpu.VMEM((DIM,), jnp.float32),
        pltpu.VMEM((CH,), jnp.float32),
        pltpu.VMEM((CH,), jnp.float32),
        pltpu.VMEM((CH,), jnp.float32),
        pltpu.VMEM((CH,), jnp.float32),
        pltpu.SemaphoreType.DMA,
        pltpu.SemaphoreType.DMA,
        pltpu.SemaphoreType.DMA,
        pltpu.SemaphoreType.DMA,
    ],
)
def _dpd_sc(x_hbm, s1_hbm, s2_hbm, perm_hbm, out_hbm,
            perm_v, s1_v, s_v, in0, in1, out0, out1,
            sem_i0, sem_i1, sem_o0, sem_o1):
    wid = lax.axis_index("s") * NC + lax.axis_index("c")
    base = wid * (ROWS_PER_W * DIM)

    pltpu.sync_copy(perm_hbm, perm_v)
    pltpu.sync_copy(s1_hbm, s1_v)
    pltpu.sync_copy(s2_hbm, s_v)

    def sbody(j, carry):
        sl = pl.ds(j * L, L)
        pv = perm_v[sl]
        s_v[sl] = plsc.load_gather(s1_v, [pv]) * s_v[sl]
        return carry

    lax.fori_loop(0, JV, sbody, 0)

    def start_in(slot, sem, c):
        pltpu.async_copy(x_hbm.at[pl.ds(base + c * CH, CH)], slot, sem)

    def start_out(slot, sem, c):
        pltpu.async_copy(slot, out_hbm.at[pl.ds(base + c * CH, CH)], sem)

    def wait_in(slot, sem):
        pltpu.make_async_copy(x_hbm.at[pl.ds(base, CH)], slot, sem).wait()

    def wait_out(slot, sem):
        pltpu.make_async_copy(slot, out_hbm.at[pl.ds(base, CH)], sem).wait()

    def compute(in_ref, out_ref):
        def jbody(jj, carry):
            for u in range(2):
                sl = pl.ds((jj * 2 + u) * L, L)
                pv = perm_v[sl]
                sv = s_v[sl]
                for r in range(R):
                    g = plsc.load_gather(in_ref, [pv + (r * DIM)])
                    out_ref[pl.ds(r * DIM + (jj * 2 + u) * L, L)] = g * sv
            return carry

        lax.fori_loop(0, JV // 2, jbody, 0)

    start_in(in0, sem_i0, 0)
    start_in(in1, sem_i1, 1)

    T = CHUNKS // 2

    def cbody(t, carry):
        wait_in(in0, sem_i0)

        @pl.when(t > 0)
        def _():
            wait_out(out0, sem_o0)

        compute(in0, out0)

        @pl.when(t < T - 1)
        def _():
            start_in(in0, sem_i0, 2 * t + 2)

        start_out(out0, sem_o0, 2 * t)

        wait_in(in1, sem_i1)

        @pl.when(t > 0)
        def _():
            wait_out(out1, sem_o1)

        compute(in1, out1)

        @pl.when(t < T - 1)
        def _():
            start_in(in1, sem_i1, 2 * t + 3)

        start_out(out1, sem_o1, 2 * t + 1)
        return carry

    lax.fori_loop(0, T, cbody, 0)

    wait_out(out0, sem_o0)
    wait_out(out1, sem_o1)


def kernel(x, sign1, sign2, perm):
    out = _dpd_sc(x.reshape(-1), sign1, sign2, perm.astype(jnp.int32))
    return out.reshape(x.shape)


# speedup vs baseline: 2.3179x; 1.9341x over previous
import functools

import jax
import jax.numpy as jnp
from jax import lax
from jax.experimental import pallas as pl
from jax.experimental.pallas import tpu as pltpu
from jax.experimental.pallas import tpu_sc as plsc

DIM = 4096
ROWS = 2 * 4096
NC = 2
NS = 16
L = 16
NW = NC * NS
ROWS_PER_W = ROWS // NW
R = 4
CH = R * DIM
CHUNKS = ROWS_PER_W // R
JV = DIM // L

_mesh = plsc.VectorSubcoreMesh(core_axis_name="c", subcore_axis_name="s")


@functools.partial(
    pl.kernel,
    mesh=_mesh,
    compiler_params=pltpu.CompilerParams(needs_layout_passes=False),
    out_type=jax.ShapeDtypeStruct((ROWS * DIM,), jnp.float32),
    scratch_types=[
        pltpu.VMEM((DIM,), jnp.int32),
        pltpu.VMEM((DIM,), jnp.float32),
        pltpu.VMEM((DIM,), jnp.float32),
        pltpu.VMEM((CH,), jnp.float32),
        pltpu.VMEM((CH,), jnp.float32),
        pltpu.VMEM((CH,), jnp.float32),
        pltpu.VMEM((CH,), jnp.float32),
        pltpu.SemaphoreType.DMA,
        pltpu.SemaphoreType.DMA,
        pltpu.SemaphoreType.DMA,
        pltpu.SemaphoreType.DMA,
    ],
)
def _dpd_sc(x_hbm, s1_hbm, s2_hbm, perm_hbm, out_hbm,
            perm_v, s1_v, s_v, in0, in1, out0, out1,
            sem_i0, sem_i1, sem_o0, sem_o1):
    wid = lax.axis_index("s") * NC + lax.axis_index("c")
    base = wid * (ROWS_PER_W * DIM)

    pltpu.sync_copy(perm_hbm, perm_v)
    pltpu.sync_copy(s1_hbm, s1_v)
    pltpu.sync_copy(s2_hbm, s_v)

    @plsc.parallel_loop(0, JV, unroll=4)
    def _sign_loop(j):
        sl = pl.ds(j * L, L)
        pv = perm_v[sl]
        s_v[sl] = plsc.load_gather(s1_v, [pv]) * s_v[sl]

    def start_in(slot, sem, c):
        pltpu.async_copy(x_hbm.at[pl.ds(base + c * CH, CH)], slot, sem)

    def start_out(slot, sem, c):
        pltpu.async_copy(slot, out_hbm.at[pl.ds(base + c * CH, CH)], sem)

    def wait_in(slot, sem):
        pltpu.make_async_copy(x_hbm.at[pl.ds(base, CH)], slot, sem).wait()

    def wait_out(slot, sem):
        pltpu.make_async_copy(slot, out_hbm.at[pl.ds(base, CH)], sem).wait()

    def compute(in_ref, out_ref):
        @plsc.parallel_loop(0, JV, unroll=8)
        def _jloop(j):
            sl = pl.ds(j * L, L)
            pv = perm_v[sl]
            sv = s_v[sl]
            for r in range(R):
                g = plsc.load_gather(in_ref, [pv + (r * DIM)])
                out_ref[pl.ds(r * DIM + j * L, L)] = g * sv

    start_in(in0, sem_i0, 0)
    start_in(in1, sem_i1, 1)

    T = CHUNKS // 2

    def cbody(t, carry):
        wait_in(in0, sem_i0)

        @pl.when(t > 0)
        def _():
            wait_out(out0, sem_o0)

        compute(in0, out0)

        @pl.when(t < T - 1)
        def _():
            start_in(in0, sem_i0, 2 * t + 2)

        start_out(out0, sem_o0, 2 * t)

        wait_in(in1, sem_i1)

        @pl.when(t > 0)
        def _():
            wait_out(out1, sem_o1)

        compute(in1, out1)

        @pl.when(t < T - 1)
        def _():
            start_in(in1, sem_i1, 2 * t + 3)

        start_out(out1, sem_o1, 2 * t + 1)
        return carry

    lax.fori_loop(0, T, cbody, 0)

    wait_out(out0, sem_o0)
    wait_out(out1, sem_o1)


def kernel(x, sign1, sign2, perm):
    out = _dpd_sc(x.reshape(-1), sign1, sign2, perm.astype(jnp.int32))
    return out.reshape(x.shape)
